# baseline (speedup 1.0000x reference)
import functools

import jax
import jax.numpy as jnp
from jax import lax
from jax.experimental import pallas as pl
from jax.experimental.pallas import tpu as pltpu
from jax.experimental.pallas import tpu_sc as plsc

NC, NS = 2, 16
NW = NC * NS
L = 16


@functools.lru_cache(maxsize=None)
def _make_feature_gather(voc: int, dim: int, seq: int, batch: int):
    feats_per_w = dim // NW
    n_vec = batch // L

    mesh = plsc.VectorSubcoreMesh(core_axis_name="c", subcore_axis_name="s")

    @functools.partial(
        pl.kernel,
        out_type=jax.ShapeDtypeStruct((seq, dim, batch), jnp.float32),
        mesh=mesh,
        scratch_types=[
            pltpu.VMEM((voc,), jnp.float32),
            pltpu.VMEM((2, batch), jnp.int32),
            pltpu.VMEM((2, batch), jnp.float32),
            pltpu.SemaphoreType.DMA,
            pltpu.SemaphoreType.DMA((2,)),
            pltpu.SemaphoreType.DMA((2,)),
        ],
        compiler_params=pltpu.CompilerParams(needs_layout_passes=False),
    )
    def gather_kernel(tab_hbm, idx_hbm, out_hbm, voc_v, idx_v, out_v, ssem,
                      isem, osem):
        wid = lax.axis_index("s") * NC + lax.axis_index("c")

        for f in range(feats_per_w):
            d = wid * feats_per_w + f
            pltpu.async_copy(tab_hbm.at[d], voc_v, ssem)
            for b in range(2):
                pltpu.async_copy(idx_hbm.at[b], idx_v.at[b], isem.at[b])
            pltpu.make_async_copy(tab_hbm.at[d], voc_v, ssem).wait()

            @pl.loop(0, seq, step=2)
            def _(s0):
                for b in range(2):
                    s = s0 + b
                    pltpu.make_async_copy(
                        idx_hbm.at[0], idx_v.at[b], isem.at[b]
                    ).wait()

                    @pl.when(s >= 2)
                    def _():
                        pltpu.make_async_copy(
                            out_v.at[b], out_hbm.at[0, 0], osem.at[b]
                        ).wait()

                    @pl.loop(0, n_vec, unroll=8)
                    def _(i):
                        g = plsc.load_gather(
                            voc_v, [idx_v[b, pl.ds(i * L, L)]]
                        )
                        out_v[b, pl.ds(i * L, L)] = g

                    @pl.when(s + 2 < seq)
                    def _():
                        pltpu.async_copy(
                            idx_hbm.at[s + 2], idx_v.at[b], isem.at[b]
                        )

                    pltpu.async_copy(out_v.at[b], out_hbm.at[s, d], osem.at[b])

            for b in range(2):
                pltpu.make_async_copy(
                    out_v.at[b], out_hbm.at[0, 0], osem.at[b]
                ).wait()

    return gather_kernel


def kernel(indices, table):
    voc, dim = table.shape
    bsz, seq = indices.shape
    tab_t = table.T
    idx_t = indices.T.astype(jnp.int32)
    out_t = _make_feature_gather(voc, dim, seq, bsz)(tab_t, idx_t)
    return jnp.transpose(out_t, (2, 0, 1))


# speedup vs baseline: 2.4538x; 2.4538x over previous
import functools

import jax
import jax.numpy as jnp
from jax import lax
from jax.experimental import pallas as pl
from jax.experimental.pallas import tpu as pltpu
from jax.experimental.pallas import tpu_sc as plsc

NC, NS = 2, 16
NW = NC * NS
L = 16


@functools.lru_cache(maxsize=None)
def _make_feature_gather(voc: int, dim: int, seq: int, batch: int):
    feats_per_w = dim // NW
    n_vec = batch // L

    mesh = plsc.VectorSubcoreMesh(core_axis_name="c", subcore_axis_name="s")

    @functools.partial(
        pl.kernel,
        out_type=jax.ShapeDtypeStruct((seq, dim, batch), jnp.float32),
        mesh=mesh,
        scratch_types=[
            pltpu.VMEM((voc,), jnp.float32),
            pltpu.VMEM((2, batch), jnp.int32),
            pltpu.VMEM((2, batch), jnp.float32),
            pltpu.SemaphoreType.DMA,
            pltpu.SemaphoreType.DMA((2,)),
            pltpu.SemaphoreType.DMA((2,)),
        ],
        compiler_params=pltpu.CompilerParams(needs_layout_passes=False),
    )
    def gather_kernel(tab_hbm, idx_hbm, out_hbm, voc_v, idx_v, out_v, ssem,
                      isem, osem):
        wid = lax.axis_index("s") * NC + lax.axis_index("c")

        for f in range(feats_per_w):
            d = wid * feats_per_w + f
            pltpu.async_copy(tab_hbm.at[d], voc_v, ssem)
            for b in range(2):
                pltpu.async_copy(idx_hbm.at[b], idx_v.at[b], isem.at[b])
            pltpu.make_async_copy(tab_hbm.at[d], voc_v, ssem).wait()

            @pl.loop(0, seq, step=2)
            def _(s0):
                for b in range(2):
                    s = s0 + b
                    pltpu.make_async_copy(
                        idx_hbm.at[0], idx_v.at[b], isem.at[b]
                    ).wait()

                    @pl.when(s >= 2)
                    def _():
                        pltpu.make_async_copy(
                            out_v.at[b], out_hbm.at[0, 0], osem.at[b]
                        ).wait()

                    @plsc.parallel_loop(0, n_vec, unroll=16)
                    def _(i):
                        g = plsc.load_gather(
                            voc_v, [idx_v[b, pl.ds(i * L, L)]]
                        )
                        out_v[b, pl.ds(i * L, L)] = g

                    @pl.when(s + 2 < seq)
                    def _():
                        pltpu.async_copy(
                            idx_hbm.at[s + 2], idx_v.at[b], isem.at[b]
                        )

                    pltpu.async_copy(out_v.at[b], out_hbm.at[s, d], osem.at[b])

            for b in range(2):
                pltpu.make_async_copy(
                    out_v.at[b], out_hbm.at[0, 0], osem.at[b]
                ).wait()

    return gather_kernel


def kernel(indices, table):
    voc, dim = table.shape
    bsz, seq = indices.shape
    tab_t = table.T
    idx_t = indices.T.astype(jnp.int32)
    out_t = _make_feature_gather(voc, dim, seq, bsz)(tab_t, idx_t)
    return jnp.transpose(out_t, (2, 0, 1))
